# baseline (speedup 1.0000x reference)
import functools

import jax
import jax.numpy as jnp
from jax import lax
from jax.experimental import pallas as pl
from jax.experimental.pallas import tpu as pltpu
from jax.experimental.pallas import tpu_sc as plsc

_NC, _NS = 2, 16
_NW = _NC * _NS
_D = 8
_NBUF = 3
_SPER = 4
_SSLOT = 2
_SPART = 4


@functools.lru_cache(maxsize=None)
def _build_permute(rows, emb, b):
    assert rows % _NW == 0 and rows % 8 == 0 and emb % _SPART == 0
    spw = rows // _NW
    clen = spw + (8 - spw % 8) % 8
    swin = clen + 16
    heb = emb // _SPART
    mesh = plsc.VectorSubcoreMesh(
        core_axis_name="c", subcore_axis_name="s",
        num_cores=_NC, num_subcores=_NS)

    t_slabs = [i for i in range(spw) if i % _SPER != _SPER - 1]
    s_slabs = [i for i in range(spw) if i % _SPER == _SPER - 1]
    nt = len(t_slabs)

    def body(src_hbm, perm_hbm, out_hbm, idx_v, shared, *rest):
        bufs = rest[:_NBUF]
        gsems = rest[_NBUF:2 * _NBUF]
        fsems = rest[2 * _NBUF:3 * _NBUF]
        sgsems = rest[3 * _NBUF:3 * _NBUF + _SSLOT]
        sfsems = rest[3 * _NBUF + _SSLOT:3 * _NBUF + 2 * _SSLOT]
        sid = lax.axis_index("s")
        wid = sid * _NC + lax.axis_index("c")
        s0 = wid * spw
        base = pl.multiple_of((s0 // 8) * 8, 8)
        off = s0 - base
        pltpu.sync_copy(perm_hbm.at[pl.ds(base, clen)], idx_v.at[pl.ds(0, clen)])

        def slab_of(i):
            return idx_v[pl.ds(off + i, 16)][0]

        def t_gather(i, s):
            pltpu.async_copy(src_hbm.at[pl.ds(slab_of(i), 1)], bufs[s],
                             gsems[s])

        def t_wait_gather(s):
            pltpu.make_async_copy(src_hbm.at[pl.ds(0, 1)], bufs[s],
                                  gsems[s]).wait()

        def t_flush(i, s):
            pltpu.async_copy(bufs[s], out_hbm.at[pl.ds(s0 + i, 1)], fsems[s])

        def t_wait_flush(s):
            pltpu.make_async_copy(bufs[s], out_hbm.at[pl.ds(s0, 1)],
                                  fsems[s]).wait()

        def sbuf(s):
            return shared.at[sid, s]

        def s_gather(i, h, s):
            pltpu.async_copy(
                src_hbm.at[pl.ds(slab_of(i), 1), pl.ds(h * heb, heb)],
                sbuf(s), sgsems[s])

        def s_wait_gather(s):
            pltpu.make_async_copy(src_hbm.at[pl.ds(0, 1), pl.ds(0, heb)],
                                  sbuf(s), sgsems[s]).wait()

        def s_flush(i, h, s):
            pltpu.async_copy(
                sbuf(s), out_hbm.at[pl.ds(s0 + i, 1), pl.ds(h * heb, heb)],
                sfsems[s])

        def s_wait_flush(s):
            pltpu.make_async_copy(
                sbuf(s), out_hbm.at[pl.ds(s0, 1), pl.ds(0, heb)],
                sfsems[s]).wait()

        parts = [(i, h) for i in s_slabs for h in range(_SPART)]

        def _a(p):
            i, h = parts[p]
            s = p % _SSLOT
            if p >= _SSLOT:
                s_wait_flush(s)
            s_gather(i, h, s)

        def _b(p):
            i, h = parts[p]
            s = p % _SSLOT
            s_wait_gather(s)
            s_flush(i, h, s)

        s_ops = []
        for p in range(len(parts)):
            s_ops.append(functools.partial(_a, p))
            if p >= _SSLOT - 1:
                s_ops.append(functools.partial(_b, p - (_SSLOT - 1)))
        for p in range(max(0, len(parts) - (_SSLOT - 1)), len(parts)):
            s_ops.append(functools.partial(_b, p))
        state = {"p": 0}

        def run_s(n):
            for _ in range(n):
                if state["p"] < len(s_ops):
                    s_ops[state["p"]]()
                    state["p"] += 1

        per_iter = -(-len(s_ops) // max(nt, 1))
        for k in range(nt + 1):
            if k < nt:
                ts = k % _NBUF
                if k >= _NBUF:
                    t_wait_flush(ts)
                t_gather(t_slabs[k], ts)
            run_s(per_iter)
            if k >= 1:
                tj = (k - 1) % _NBUF
                t_wait_gather(tj)
                t_flush(t_slabs[k - 1], tj)
        run_s(len(s_ops))
        for k in range(max(0, nt - _NBUF), nt):
            t_wait_flush(k % _NBUF)
        for p in range(max(0, len(parts) - _SSLOT), len(parts)):
            s_wait_flush(p % _SSLOT)

    return pl.kernel(
        body,
        out_type=jax.ShapeDtypeStruct((rows, emb, b), jnp.float32),
        mesh=mesh,
        scratch_types=(
            [pltpu.VMEM((swin,), jnp.int32),
             pltpu.VMEM_SHARED((_NS, _SSLOT, 1, heb, b), jnp.float32)]
            + [pltpu.VMEM((1, emb, b), jnp.float32)] * _NBUF
            + [pltpu.SemaphoreType.DMA] * (2 * _NBUF + 2 * _SSLOT)
        ),
    )


def kernel(M, permutator):
    b, n_cond, n_col, emb = M.shape
    rows = n_cond * n_col
    Mv = jnp.transpose(M.reshape(b, rows, emb), (1, 2, 0))
    perm = permutator.astype(jnp.int32)
    out_v = _build_permute(rows, emb, b)(Mv, perm)
    return jnp.transpose(out_v.reshape(rows // _D, _D, emb, b), (3, 0, 1, 2))


# speedup vs baseline: 1.0143x; 1.0143x over previous
import functools

import jax
import jax.numpy as jnp
from jax import lax
from jax.experimental import pallas as pl
from jax.experimental.pallas import tpu as pltpu
from jax.experimental.pallas import tpu_sc as plsc

_NC, _NS = 2, 16
_NW = _NC * _NS
_D = 8
_NBUF = 3
_SPER = 4
_SSLOT = 1
_SPART = 2


@functools.lru_cache(maxsize=None)
def _build_permute(rows, emb, b):
    assert rows % _NW == 0 and rows % 8 == 0 and emb % _SPART == 0
    spw = rows // _NW
    clen = spw + (8 - spw % 8) % 8
    swin = clen + 16
    heb = emb // _SPART
    mesh = plsc.VectorSubcoreMesh(
        core_axis_name="c", subcore_axis_name="s",
        num_cores=_NC, num_subcores=_NS)

    t_slabs = [i for i in range(spw) if i % _SPER != _SPER - 1]
    s_slabs = [i for i in range(spw) if i % _SPER == _SPER - 1]
    nt = len(t_slabs)

    def body(src_hbm, perm_hbm, out_hbm, idx_v, shared, *rest):
        bufs = rest[:_NBUF]
        gsems = rest[_NBUF:2 * _NBUF]
        fsems = rest[2 * _NBUF:3 * _NBUF]
        sgsems = rest[3 * _NBUF:3 * _NBUF + _SSLOT]
        sfsems = rest[3 * _NBUF + _SSLOT:3 * _NBUF + 2 * _SSLOT]
        sid = lax.axis_index("s")
        wid = sid * _NC + lax.axis_index("c")
        s0 = wid * spw
        base = pl.multiple_of((s0 // 8) * 8, 8)
        off = s0 - base
        pltpu.sync_copy(perm_hbm.at[pl.ds(base, clen)], idx_v.at[pl.ds(0, clen)])

        def slab_of(i):
            return idx_v[pl.ds(off + i, 16)][0]

        def t_gather(i, s):
            pltpu.async_copy(src_hbm.at[pl.ds(slab_of(i), 1)], bufs[s],
                             gsems[s])

        def t_wait_gather(s):
            pltpu.make_async_copy(src_hbm.at[pl.ds(0, 1)], bufs[s],
                                  gsems[s]).wait()

        def t_flush(i, s):
            pltpu.async_copy(bufs[s], out_hbm.at[pl.ds(s0 + i, 1)], fsems[s])

        def t_wait_flush(s):
            pltpu.make_async_copy(bufs[s], out_hbm.at[pl.ds(s0, 1)],
                                  fsems[s]).wait()

        def sbuf(s):
            return shared.at[sid, s]

        def s_gather(i, h, s):
            pltpu.async_copy(
                src_hbm.at[pl.ds(slab_of(i), 1), pl.ds(h * heb, heb)],
                sbuf(s), sgsems[s])

        def s_wait_gather(s):
            pltpu.make_async_copy(src_hbm.at[pl.ds(0, 1), pl.ds(0, heb)],
                                  sbuf(s), sgsems[s]).wait()

        def s_flush(i, h, s):
            pltpu.async_copy(
                sbuf(s), out_hbm.at[pl.ds(s0 + i, 1), pl.ds(h * heb, heb)],
                sfsems[s])

        def s_wait_flush(s):
            pltpu.make_async_copy(
                sbuf(s), out_hbm.at[pl.ds(s0, 1), pl.ds(0, heb)],
                sfsems[s]).wait()

        parts = [(i, h) for i in s_slabs for h in range(_SPART)]

        def _a(p):
            i, h = parts[p]
            s = p % _SSLOT
            if p >= _SSLOT:
                s_wait_flush(s)
            s_gather(i, h, s)

        def _b(p):
            i, h = parts[p]
            s = p % _SSLOT
            s_wait_gather(s)
            s_flush(i, h, s)

        s_ops = []
        for p in range(len(parts)):
            s_ops.append(functools.partial(_a, p))
            if p >= _SSLOT - 1:
                s_ops.append(functools.partial(_b, p - (_SSLOT - 1)))
        for p in range(max(0, len(parts) - (_SSLOT - 1)), len(parts)):
            s_ops.append(functools.partial(_b, p))
        state = {"p": 0}

        def run_s(n):
            for _ in range(n):
                if state["p"] < len(s_ops):
                    s_ops[state["p"]]()
                    state["p"] += 1

        per_iter = -(-len(s_ops) // max(nt, 1))
        for k in range(nt + 1):
            if k < nt:
                ts = k % _NBUF
                if k >= _NBUF:
                    t_wait_flush(ts)
                t_gather(t_slabs[k], ts)
            run_s(per_iter)
            if k >= 1:
                tj = (k - 1) % _NBUF
                t_wait_gather(tj)
                t_flush(t_slabs[k - 1], tj)
        run_s(len(s_ops))
        for k in range(max(0, nt - _NBUF), nt):
            t_wait_flush(k % _NBUF)
        for p in range(max(0, len(parts) - _SSLOT), len(parts)):
            s_wait_flush(p % _SSLOT)

    return pl.kernel(
        body,
        out_type=jax.ShapeDtypeStruct((rows, emb, b), jnp.float32),
        mesh=mesh,
        scratch_types=(
            [pltpu.VMEM((swin,), jnp.int32),
             pltpu.VMEM_SHARED((_NS, _SSLOT, 1, heb, b), jnp.float32)]
            + [pltpu.VMEM((1, emb, b), jnp.float32)] * _NBUF
            + [pltpu.SemaphoreType.DMA] * (2 * _NBUF + 2 * _SSLOT)
        ),
    )


def kernel(M, permutator):
    b, n_cond, n_col, emb = M.shape
    rows = n_cond * n_col
    Mv = jnp.transpose(M.reshape(b, rows, emb), (1, 2, 0))
    perm = permutator.astype(jnp.int32)
    out_v = _build_permute(rows, emb, b)(Mv, perm)
    return jnp.transpose(out_v.reshape(rows // _D, _D, emb, b), (3, 0, 1, 2))
